# baseline (speedup 1.0000x reference)
import functools

import jax
import jax.numpy as jnp
from jax import lax
from jax.experimental import pallas as pl
from jax.experimental.pallas import tpu as pltpu
from jax.experimental.pallas import tpu_sc as plsc

N_NODES = 10000
D = 128
NPAD = 10240
E = 320000
NW = 32
CHUNK = 32
KCH = 320
KBLK = 16
NBLK = KCH // KBLK
NBUF = 8
DCHUNK = 128
DKCH = 80
E_PAD = NW * KCH * CHUNK
TSTEPS = 10
BATCH = 1000
HID = 128
HID2 = 64
G4 = 4 * HID

_sc_mesh = plsc.VectorSubcoreMesh(core_axis_name="c", subcore_axis_name="s")


def _sc_deg_body(dst_hbm, deg_hbm, idx_v, ones_v, zero_v, hist_sh, sd):
    cid = lax.axis_index("c")
    sid = lax.axis_index("s")
    wid = cid * 16 + sid

    ones16 = jnp.ones((16,), jnp.float32)
    zeros16 = jnp.zeros((16,), jnp.float32)
    for i in range(8):
        ones_v[pl.ds(i * 16, 16)] = ones16

    def _zfill(i, _):
        zero_v[pl.ds(i * 16, 16)] = zeros16
        return 0
    lax.fori_loop(0, 40, _zfill, 0)

    pltpu.sync_copy(zero_v, hist_sh.at[pl.ds(sid * 640, 640)])
    pltpu.sync_copy(dst_hbm.at[wid], idx_v)
    plsc.subcore_barrier()

    def _scat(j, _):
        pltpu.async_copy(ones_v, hist_sh.at[idx_v.at[j]], sd, add=True)
        return 0
    lax.fori_loop(0, DKCH, _scat, 0)

    def _drain(j, _):
        pltpu.make_async_copy(ones_v, hist_sh.at[idx_v.at[0]], sd).wait()
        return 0
    lax.fori_loop(0, DKCH, _drain, 0)

    plsc.subcore_barrier()
    pltpu.sync_copy(hist_sh.at[pl.ds(sid * 640, 640)],
                    deg_hbm.at[cid, pl.ds(sid * 640, 640)])


_sc_deg = functools.partial(
    pl.kernel,
    out_type=jax.ShapeDtypeStruct((2, NPAD), jnp.float32),
    mesh=_sc_mesh,
    scratch_types=[
        pltpu.VMEM((DKCH, DCHUNK), jnp.int32),
        pltpu.VMEM((DCHUNK,), jnp.float32),
        pltpu.VMEM((640,), jnp.float32),
        pltpu.VMEM_SHARED((NPAD,), jnp.float32),
        pltpu.SemaphoreType.DMA,
    ],
)(_sc_deg_body)


def _sc_agg_body(y_hbm, src_hbm, dst_hbm, z_hbm,
                 sA, dA, sB, dB, r0, r1, r2, r3, r4, r5, r6, r7,
                 zbuf_v, acc_sh, g0, g1, g2, g3, g4, g5, g6, g7, ib):
    cid = lax.axis_index("c")
    sid = lax.axis_index("s")
    wid = cid * 16 + sid
    idxsets = ((sA, dA), (sB, dB))
    rows = (r0, r1, r2, r3, r4, r5, r6, r7)
    gsems = (g0, g1, g2, g3, g4, g5, g6, g7)

    zeros16 = jnp.zeros((16,), jnp.float32)

    def _zfill(i, _):
        for k in range(8):
            zbuf_v[i, pl.ds(k * 16, 16)] = zeros16
        return 0
    lax.fori_loop(0, 16, _zfill, 0)

    for t in range(40):
        pltpu.sync_copy(zbuf_v, acc_sh.at[pl.ds(sid * 640 + t * 16, 16)])

    def _stage(bset, blk):
        s_, d_ = idxsets[bset]
        pltpu.async_copy(src_hbm.at[wid, pl.ds(blk * KBLK, KBLK)], s_, ib)
        pltpu.async_copy(dst_hbm.at[wid, pl.ds(blk * KBLK, KBLK)], d_, ib)

    def _stage_wait(bset, blk):
        s_, d_ = idxsets[bset]
        pltpu.make_async_copy(
            src_hbm.at[wid, pl.ds(blk * KBLK, KBLK)], s_, ib).wait()
        pltpu.make_async_copy(
            dst_hbm.at[wid, pl.ds(blk * KBLK, KBLK)], d_, ib).wait()

    def _g(s_, i, rb, gs):
        pltpu.async_copy(y_hbm.at[s_.at[i]], rb, gs)

    def _wg(s_, i, rb, gs):
        pltpu.make_async_copy(y_hbm.at[s_.at[i]], rb, gs).wait()

    def _s(d_, i, rb):
        pltpu.sync_copy(rb, acc_sh.at[d_.at[i]], add=True)

    _stage(0, 0)
    _stage_wait(0, 0)
    _stage(1, 1)
    plsc.subcore_barrier()

    sa, da = idxsets[0]
    for t in range(NBUF):
        _g(sa, t, rows[t], gsems[t])

    for b in range(NBLK):
        s_, d_ = idxsets[b % 2]

        def _inner(i, _, s_=s_, d_=d_):
            j0 = NBUF * i
            for t in range(NBUF):
                _wg(s_, j0 + t, rows[t], gsems[t])
                _s(d_, j0 + t, rows[t])
                _g(s_, j0 + t + NBUF, rows[t], gsems[t])
            return 0
        lax.fori_loop(0, KBLK // NBUF - 1, _inner, 0)

        j0 = KBLK - NBUF
        if b < NBLK - 1:
            nset = (b + 1) % 2
            _stage_wait(nset, b + 1)
            sn, _dn = idxsets[nset]
            for t in range(NBUF):
                _wg(s_, j0 + t, rows[t], gsems[t])
                _s(d_, j0 + t, rows[t])
                _g(sn, t, rows[t], gsems[t])
            if b < NBLK - 2:
                _stage(b % 2, b + 2)
        else:
            for t in range(NBUF):
                _wg(s_, j0 + t, rows[t], gsems[t])
                _s(d_, j0 + t, rows[t])

    plsc.subcore_barrier()
    pltpu.sync_copy(acc_sh.at[pl.ds(sid * 640, 640)],
                    z_hbm.at[cid, pl.ds(sid * 640, 640)])


_sc_agg = functools.partial(
    pl.kernel,
    out_type=jax.ShapeDtypeStruct((2, NPAD, D), jnp.float32),
    mesh=_sc_mesh,
    scratch_types=[
        pltpu.VMEM((KBLK, CHUNK), jnp.int32),
        pltpu.VMEM((KBLK, CHUNK), jnp.int32),
        pltpu.VMEM((KBLK, CHUNK), jnp.int32),
        pltpu.VMEM((KBLK, CHUNK), jnp.int32),
        pltpu.VMEM((CHUNK, D), jnp.float32),
        pltpu.VMEM((CHUNK, D), jnp.float32),
        pltpu.VMEM((CHUNK, D), jnp.float32),
        pltpu.VMEM((CHUNK, D), jnp.float32),
        pltpu.VMEM((CHUNK, D), jnp.float32),
        pltpu.VMEM((CHUNK, D), jnp.float32),
        pltpu.VMEM((CHUNK, D), jnp.float32),
        pltpu.VMEM((CHUNK, D), jnp.float32),
        pltpu.VMEM((16, D), jnp.float32),
        pltpu.VMEM_SHARED((NPAD, D), jnp.float32),
        pltpu.SemaphoreType.DMA,
        pltpu.SemaphoreType.DMA,
        pltpu.SemaphoreType.DMA,
        pltpu.SemaphoreType.DMA,
        pltpu.SemaphoreType.DMA,
        pltpu.SemaphoreType.DMA,
        pltpu.SemaphoreType.DMA,
        pltpu.SemaphoreType.DMA,
        pltpu.SemaphoreType.DMA,
    ],
)(_sc_agg_body)


def _scale_body(x_ref, d_ref, o_ref):
    i = pl.program_id(0)
    row = i * 1280 + lax.broadcasted_iota(jnp.int32, (1280, D), 0)
    o_ref[...] = jnp.where(row < N_NODES, x_ref[...] * d_ref[...], 0.0)


def _tc_scale(x, dinv_col):
    return pl.pallas_call(
        _scale_body,
        grid=(8,),
        in_specs=[
            pl.BlockSpec((1280, D), lambda i: (i, 0)),
            pl.BlockSpec((1280, 1), lambda i: (i, 0)),
        ],
        out_specs=pl.BlockSpec((1280, D), lambda i: (i, 0)),
        out_shape=jax.ShapeDtypeStruct((NPAD, D), jnp.float32),
    )(x, dinv_col)


def _head_body(z_ref, y_ref, d_ref, w_ref, b_ref, wih_ref, whh_ref, bs_ref,
               f1w_ref, f1b_ref, f2w_ref, f2b_ref, o_ref, gx_ref):
    i = pl.program_id(0)

    @pl.when(i < 8)
    def _gcn():
        zsum = z_ref[0] + z_ref[1] + y_ref[...]
        agg = zsum * d_ref[...]
        g = jnp.dot(agg, w_ref[...], preferred_element_type=jnp.float32)
        g = g + b_ref[...]
        h = jax.nn.relu(g[:, :HID]) + jax.nn.relu(g[:, HID:])
        gx_ref[pl.ds(i * 1280, 1280), :] = jnp.dot(
            h, wih_ref[...], preferred_element_type=jnp.float32)

    @pl.when(i == 8)
    def _lstm():
        whh = whh_ref[...]
        bs = bs_ref[...]

        def _sigm(v):
            return 1.0 / (1.0 + jnp.exp(-v))

        def _step(t, carry):
            hp, cp = carry
            gates = gx_ref[pl.ds(t * BATCH, BATCH), :]
            gates = gates + jnp.dot(hp, whh,
                                    preferred_element_type=jnp.float32)
            gates = gates + bs
            ig = _sigm(gates[:, 0 * HID:1 * HID])
            fg = _sigm(gates[:, 1 * HID:2 * HID])
            gg = jnp.tanh(gates[:, 2 * HID:3 * HID])
            og = _sigm(gates[:, 3 * HID:4 * HID])
            c = fg * cp + ig * gg
            hn = og * jnp.tanh(c)
            return (hn, c)

        h0 = jnp.zeros((BATCH, HID), jnp.float32)
        c0 = jnp.zeros((BATCH, HID), jnp.float32)
        hT, _ = lax.fori_loop(0, TSTEPS, _step, (h0, c0))
        u = jax.nn.relu(
            jnp.dot(hT, f1w_ref[...], preferred_element_type=jnp.float32)
            + f1b_ref[...])
        o_ref[...] = jnp.dot(u, f2w_ref[...],
                             preferred_element_type=jnp.float32) + f2b_ref[...]


def _tc_head(z, y, dinv_col, W12t, b12, Wiht, Whht, bsum,
             f1w, f1b, f2w, f2b):
    return pl.pallas_call(
        _head_body,
        grid=(9,),
        in_specs=[
            pl.BlockSpec((2, 1280, D), lambda i: (0, i % 8, 0)),
            pl.BlockSpec((1280, D), lambda i: (i % 8, 0)),
            pl.BlockSpec((1280, 1), lambda i: (i % 8, 0)),
            pl.BlockSpec((D, 2 * HID), lambda i: (0, 0)),
            pl.BlockSpec((1, 2 * HID), lambda i: (0, 0)),
            pl.BlockSpec((D, G4), lambda i: (0, 0)),
            pl.BlockSpec((D, G4), lambda i: (0, 0)),
            pl.BlockSpec((1, G4), lambda i: (0, 0)),
            pl.BlockSpec((D, HID2), lambda i: (0, 0)),
            pl.BlockSpec((1, HID2), lambda i: (0, 0)),
            pl.BlockSpec((HID2, D), lambda i: (0, 0)),
            pl.BlockSpec((1, D), lambda i: (0, 0)),
        ],
        out_specs=pl.BlockSpec((BATCH, HID), lambda i: (0, 0)),
        out_shape=jax.ShapeDtypeStruct((BATCH, HID), jnp.float32),
        scratch_shapes=[pltpu.VMEM((NPAD, G4), jnp.float32)],
    )(z, y, dinv_col, W12t, b12, Wiht, Whht, bsum, f1w, f1b, f2w, f2b)


def kernel(x, edge_index, W1, b1, W2, b2, Wih, Whh, bih, bhh,
           fc1W, fc1b, fc2W, fc2b):
    n_fill = E_PAD - E
    pad_idx = (N_NODES
               + jnp.arange(n_fill, dtype=jnp.int32) % (NPAD - N_NODES))
    src_flat = jnp.concatenate([edge_index[0], pad_idx])
    dst_flat = jnp.concatenate([edge_index[1], pad_idx])
    src_p = src_flat.reshape(NW, KCH, CHUNK)
    dst_p = dst_flat.reshape(NW, KCH, CHUNK)

    degp = _sc_deg(dst_flat.reshape(NW, DKCH, DCHUNK))
    deg = degp[0] + degp[1] + 1.0
    dinv_col = lax.rsqrt(deg)[:, None]

    y = _tc_scale(x, dinv_col)

    z = _sc_agg(y, src_p, dst_p)

    W12t = jnp.concatenate([W1.T, W2.T], axis=1)
    b12 = jnp.concatenate([b1, b2])[None]

    return _tc_head(z, y, dinv_col, W12t, b12, Wih.T, Whh.T,
                    (bih + bhh)[None], fc1W.T, fc1b[None],
                    fc2W.T, fc2b[None])


# speedup vs baseline: 1.0106x; 1.0106x over previous
import functools

import jax
import jax.numpy as jnp
from jax import lax
from jax.experimental import pallas as pl
from jax.experimental.pallas import tpu as pltpu
from jax.experimental.pallas import tpu_sc as plsc

N_NODES = 10000
D = 128
NPAD = 10240
E = 320000
NW = 32
CHUNK = 64
KCH = 160
KBLK = 16
NBLK = KCH // KBLK
NBUF = 4
DCHUNK = 128
DKCH = 80
E_PAD = NW * KCH * CHUNK
TSTEPS = 10
BATCH = 1000
HID = 128
HID2 = 64
G4 = 4 * HID

_sc_mesh = plsc.VectorSubcoreMesh(core_axis_name="c", subcore_axis_name="s")


def _sc_deg_body(dst_hbm, deg_hbm, idx_v, ones_v, zero_v, hist_sh, sd):
    cid = lax.axis_index("c")
    sid = lax.axis_index("s")
    wid = cid * 16 + sid

    ones16 = jnp.ones((16,), jnp.float32)
    zeros16 = jnp.zeros((16,), jnp.float32)
    for i in range(8):
        ones_v[pl.ds(i * 16, 16)] = ones16

    def _zfill(i, _):
        zero_v[pl.ds(i * 16, 16)] = zeros16
        return 0
    lax.fori_loop(0, 40, _zfill, 0)

    pltpu.sync_copy(zero_v, hist_sh.at[pl.ds(sid * 640, 640)])
    pltpu.sync_copy(dst_hbm.at[wid], idx_v)
    plsc.subcore_barrier()

    def _scat(j, _):
        pltpu.async_copy(ones_v, hist_sh.at[idx_v.at[j]], sd, add=True)
        return 0
    lax.fori_loop(0, DKCH, _scat, 0)

    def _drain(j, _):
        pltpu.make_async_copy(ones_v, hist_sh.at[idx_v.at[0]], sd).wait()
        return 0
    lax.fori_loop(0, DKCH, _drain, 0)

    plsc.subcore_barrier()
    pltpu.sync_copy(hist_sh.at[pl.ds(sid * 640, 640)],
                    deg_hbm.at[cid, pl.ds(sid * 640, 640)])


_sc_deg = functools.partial(
    pl.kernel,
    out_type=jax.ShapeDtypeStruct((2, NPAD), jnp.float32),
    mesh=_sc_mesh,
    scratch_types=[
        pltpu.VMEM((DKCH, DCHUNK), jnp.int32),
        pltpu.VMEM((DCHUNK,), jnp.float32),
        pltpu.VMEM((640,), jnp.float32),
        pltpu.VMEM_SHARED((NPAD,), jnp.float32),
        pltpu.SemaphoreType.DMA,
    ],
)(_sc_deg_body)


def _sc_agg_body(y_hbm, src_hbm, dst_hbm, z_hbm,
                 sA, dA, sB, dB, r0, r1, r2, r3, zbuf_v, acc_sh,
                 g0, g1, g2, g3, ib):
    cid = lax.axis_index("c")
    sid = lax.axis_index("s")
    wid = cid * 16 + sid
    idxsets = ((sA, dA), (sB, dB))
    rows = (r0, r1, r2, r3)
    gsems = (g0, g1, g2, g3)

    zeros16 = jnp.zeros((16,), jnp.float32)

    def _zfill(i, _):
        for k in range(8):
            zbuf_v[i, pl.ds(k * 16, 16)] = zeros16
        return 0
    lax.fori_loop(0, 16, _zfill, 0)

    for t in range(40):
        pltpu.sync_copy(zbuf_v, acc_sh.at[pl.ds(sid * 640 + t * 16, 16)])

    def _stage(bset, blk):
        s_, d_ = idxsets[bset]
        pltpu.async_copy(src_hbm.at[wid, pl.ds(blk * KBLK, KBLK)], s_, ib)
        pltpu.async_copy(dst_hbm.at[wid, pl.ds(blk * KBLK, KBLK)], d_, ib)

    def _stage_wait(bset, blk):
        s_, d_ = idxsets[bset]
        pltpu.make_async_copy(
            src_hbm.at[wid, pl.ds(blk * KBLK, KBLK)], s_, ib).wait()
        pltpu.make_async_copy(
            dst_hbm.at[wid, pl.ds(blk * KBLK, KBLK)], d_, ib).wait()

    def _g(s_, i, rb, gs):
        pltpu.async_copy(y_hbm.at[s_.at[i]], rb, gs)

    def _wg(s_, i, rb, gs):
        pltpu.make_async_copy(y_hbm.at[s_.at[i]], rb, gs).wait()

    def _s(d_, i, rb):
        pltpu.sync_copy(rb, acc_sh.at[d_.at[i]], add=True)

    _stage(0, 0)
    _stage_wait(0, 0)
    _stage(1, 1)
    plsc.subcore_barrier()

    sa, da = idxsets[0]
    for t in range(NBUF):
        _g(sa, t, rows[t], gsems[t])

    for b in range(NBLK):
        s_, d_ = idxsets[b % 2]

        def _inner(i, _, s_=s_, d_=d_):
            j0 = NBUF * i
            for t in range(NBUF):
                _wg(s_, j0 + t, rows[t], gsems[t])
                _s(d_, j0 + t, rows[t])
                _g(s_, j0 + t + NBUF, rows[t], gsems[t])
            return 0
        lax.fori_loop(0, KBLK // NBUF - 1, _inner, 0)

        j0 = KBLK - NBUF
        if b < NBLK - 1:
            nset = (b + 1) % 2
            _stage_wait(nset, b + 1)
            sn, _dn = idxsets[nset]
            for t in range(NBUF):
                _wg(s_, j0 + t, rows[t], gsems[t])
                _s(d_, j0 + t, rows[t])
                _g(sn, t, rows[t], gsems[t])
            if b < NBLK - 2:
                _stage(b % 2, b + 2)
        else:
            for t in range(NBUF):
                _wg(s_, j0 + t, rows[t], gsems[t])
                _s(d_, j0 + t, rows[t])

    plsc.subcore_barrier()
    pltpu.sync_copy(acc_sh.at[pl.ds(sid * 640, 640)],
                    z_hbm.at[cid, pl.ds(sid * 640, 640)])


_sc_agg = functools.partial(
    pl.kernel,
    out_type=jax.ShapeDtypeStruct((2, NPAD, D), jnp.float32),
    mesh=_sc_mesh,
    scratch_types=[
        pltpu.VMEM((KBLK, CHUNK), jnp.int32),
        pltpu.VMEM((KBLK, CHUNK), jnp.int32),
        pltpu.VMEM((KBLK, CHUNK), jnp.int32),
        pltpu.VMEM((KBLK, CHUNK), jnp.int32),
        pltpu.VMEM((CHUNK, D), jnp.float32),
        pltpu.VMEM((CHUNK, D), jnp.float32),
        pltpu.VMEM((CHUNK, D), jnp.float32),
        pltpu.VMEM((CHUNK, D), jnp.float32),
        pltpu.VMEM((16, D), jnp.float32),
        pltpu.VMEM_SHARED((NPAD, D), jnp.float32),
        pltpu.SemaphoreType.DMA,
        pltpu.SemaphoreType.DMA,
        pltpu.SemaphoreType.DMA,
        pltpu.SemaphoreType.DMA,
        pltpu.SemaphoreType.DMA,
    ],
)(_sc_agg_body)


def _scale_body(x_ref, d_ref, o_ref):
    i = pl.program_id(0)
    row = i * 1280 + lax.broadcasted_iota(jnp.int32, (1280, D), 0)
    o_ref[...] = jnp.where(row < N_NODES, x_ref[...] * d_ref[...], 0.0)


def _tc_scale(x, dinv_col):
    return pl.pallas_call(
        _scale_body,
        grid=(8,),
        in_specs=[
            pl.BlockSpec((1280, D), lambda i: (i, 0)),
            pl.BlockSpec((1280, 1), lambda i: (i, 0)),
        ],
        out_specs=pl.BlockSpec((1280, D), lambda i: (i, 0)),
        out_shape=jax.ShapeDtypeStruct((NPAD, D), jnp.float32),
    )(x, dinv_col)


def _dotT(a, b):
    return lax.dot_general(a, b, (((1,), (1,)), ((), ())),
                           preferred_element_type=jnp.float32)


def _head_body(z_ref, y_ref, d_ref, w1_ref, b1_ref, w2_ref, b2_ref,
               wih_ref, whh_ref, bs_ref,
               f1w_ref, f1b_ref, f2w_ref, f2b_ref, o_ref, gx_ref):
    i = pl.program_id(0)

    @pl.when(i < 8)
    def _gcn():
        zsum = z_ref[0] + z_ref[1] + y_ref[...]
        agg = zsum * d_ref[...]
        g1 = _dotT(agg, w1_ref[...]) + b1_ref[...]
        g2 = _dotT(agg, w2_ref[...]) + b2_ref[...]
        h = jax.nn.relu(g1) + jax.nn.relu(g2)
        gx_ref[pl.ds(i * 1280, 1280), :] = _dotT(h, wih_ref[...])

    @pl.when(i == 8)
    def _lstm():
        whh = whh_ref[...]
        bs = bs_ref[...]

        def _sigm(v):
            return 1.0 / (1.0 + jnp.exp(-v))

        def _step(t, carry):
            hp, cp = carry
            gates = gx_ref[pl.ds(t * BATCH, BATCH), :]
            gates = gates + _dotT(hp, whh)
            gates = gates + bs
            ig = _sigm(gates[:, 0 * HID:1 * HID])
            fg = _sigm(gates[:, 1 * HID:2 * HID])
            gg = jnp.tanh(gates[:, 2 * HID:3 * HID])
            og = _sigm(gates[:, 3 * HID:4 * HID])
            c = fg * cp + ig * gg
            hn = og * jnp.tanh(c)
            return (hn, c)

        h0 = jnp.zeros((BATCH, HID), jnp.float32)
        c0 = jnp.zeros((BATCH, HID), jnp.float32)
        hT, _ = lax.fori_loop(0, TSTEPS, _step, (h0, c0))
        u = jax.nn.relu(_dotT(hT, f1w_ref[...]) + f1b_ref[...])
        o_ref[...] = _dotT(u, f2w_ref[...]) + f2b_ref[...]


def _tc_head(z, y, dinv_col, W1, b1, W2, b2, Wih, Whh, bsum,
             fc1W, fc1b, fc2W, fc2b):
    return pl.pallas_call(
        _head_body,
        grid=(9,),
        in_specs=[
            pl.BlockSpec((2, 1280, D), lambda i: (0, i % 8, 0)),
            pl.BlockSpec((1280, D), lambda i: (i % 8, 0)),
            pl.BlockSpec((1280, 1), lambda i: (i % 8, 0)),
            pl.BlockSpec((HID, D), lambda i: (0, 0)),
            pl.BlockSpec((1, HID), lambda i: (0, 0)),
            pl.BlockSpec((HID, D), lambda i: (0, 0)),
            pl.BlockSpec((1, HID), lambda i: (0, 0)),
            pl.BlockSpec((G4, HID), lambda i: (0, 0)),
            pl.BlockSpec((G4, HID), lambda i: (0, 0)),
            pl.BlockSpec((1, G4), lambda i: (0, 0)),
            pl.BlockSpec((HID2, HID), lambda i: (0, 0)),
            pl.BlockSpec((1, HID2), lambda i: (0, 0)),
            pl.BlockSpec((HID, HID2), lambda i: (0, 0)),
            pl.BlockSpec((1, D), lambda i: (0, 0)),
        ],
        out_specs=pl.BlockSpec((BATCH, HID), lambda i: (0, 0)),
        out_shape=jax.ShapeDtypeStruct((BATCH, HID), jnp.float32),
        scratch_shapes=[pltpu.VMEM((NPAD, G4), jnp.float32)],
    )(z, y, dinv_col, W1, b1, W2, b2, Wih, Whh, bsum,
      fc1W, fc1b, fc2W, fc2b)


def kernel(x, edge_index, W1, b1, W2, b2, Wih, Whh, bih, bhh,
           fc1W, fc1b, fc2W, fc2b):
    n_fill = E_PAD - E
    pad_idx = (N_NODES
               + jnp.arange(n_fill, dtype=jnp.int32) % (NPAD - N_NODES))
    src_flat = jnp.concatenate([edge_index[0], pad_idx])
    dst_flat = jnp.concatenate([edge_index[1], pad_idx])
    src_p = src_flat.reshape(NW, KCH, CHUNK)
    dst_p = dst_flat.reshape(NW, KCH, CHUNK)

    degp = _sc_deg(dst_flat.reshape(NW, DKCH, DCHUNK))
    deg = degp[0] + degp[1] + 1.0
    dinv_col = lax.rsqrt(deg)[:, None]

    y = _tc_scale(x, dinv_col)

    z = _sc_agg(y, src_p, dst_p)

    return _tc_head(z, y, dinv_col, W1, b1[None], W2, b2[None],
                    Wih, Whh, (bih + bhh)[None],
                    fc1W, fc1b[None], fc2W, fc2b[None])


# speedup vs baseline: 1.0376x; 1.0267x over previous
import functools

import jax
import jax.numpy as jnp
from jax import lax
from jax.experimental import pallas as pl
from jax.experimental.pallas import tpu as pltpu
from jax.experimental.pallas import tpu_sc as plsc

N_NODES = 10000
D = 128
NPAD = 10240
E = 320000
NW = 32
CHUNK = 64
KCH = 160
KBLK = 16
NBLK = KCH // KBLK
NBUF = 4
DCHUNK = 128
DKCH = 80
E_PAD = NW * KCH * CHUNK
TSTEPS = 10
BATCH = 1000
HID = 128
HID2 = 64
G4 = 4 * HID

_sc_mesh = plsc.VectorSubcoreMesh(core_axis_name="c", subcore_axis_name="s")


def _sc_deg_body(dst_hbm, deg_hbm, idx_v, ones_v, zero_v, hist_sh, sd):
    cid = lax.axis_index("c")
    sid = lax.axis_index("s")
    wid = cid * 16 + sid

    ones16 = jnp.ones((16,), jnp.float32)
    zeros16 = jnp.zeros((16,), jnp.float32)
    for i in range(8):
        ones_v[pl.ds(i * 16, 16)] = ones16

    def _zfill(i, _):
        zero_v[pl.ds(i * 16, 16)] = zeros16
        return 0
    lax.fori_loop(0, 40, _zfill, 0)

    pltpu.sync_copy(zero_v, hist_sh.at[pl.ds(sid * 640, 640)])
    pltpu.sync_copy(dst_hbm.at[wid], idx_v)
    plsc.subcore_barrier()

    def _scat(j, _):
        pltpu.async_copy(ones_v, hist_sh.at[idx_v.at[j]], sd, add=True)
        return 0
    lax.fori_loop(0, DKCH, _scat, 0)

    def _drain(j, _):
        pltpu.make_async_copy(ones_v, hist_sh.at[idx_v.at[0]], sd).wait()
        return 0
    lax.fori_loop(0, DKCH, _drain, 0)

    plsc.subcore_barrier()
    pltpu.sync_copy(hist_sh.at[pl.ds(sid * 640, 640)],
                    deg_hbm.at[cid, pl.ds(sid * 640, 640)])


_sc_deg = functools.partial(
    pl.kernel,
    out_type=jax.ShapeDtypeStruct((2, NPAD), jnp.float32),
    mesh=_sc_mesh,
    scratch_types=[
        pltpu.VMEM((DKCH, DCHUNK), jnp.int32),
        pltpu.VMEM((DCHUNK,), jnp.float32),
        pltpu.VMEM((640,), jnp.float32),
        pltpu.VMEM_SHARED((NPAD,), jnp.float32),
        pltpu.SemaphoreType.DMA,
    ],
)(_sc_deg_body)


def _sc_agg_body(y_hbm, src_hbm, dst_hbm, z_hbm,
                 sA, dA, sB, dB, r0, r1, r2, r3, zbuf_v, acc_sh,
                 g0, g1, g2, g3, ib, zs):
    cid = lax.axis_index("c")
    sid = lax.axis_index("s")
    wid = cid * 16 + sid
    idxsets = ((sA, dA), (sB, dB))
    rows = (r0, r1, r2, r3)
    gsems = (g0, g1, g2, g3)

    zeros16 = jnp.zeros((16,), jnp.float32)

    def _zfill(i, _):
        for k in range(8):
            zbuf_v[i, pl.ds(k * 16, 16)] = zeros16
        return 0
    lax.fori_loop(0, 16, _zfill, 0)

    for t in range(40):
        pltpu.async_copy(zbuf_v, acc_sh.at[pl.ds(sid * 640 + t * 16, 16)], zs)

    def _stage(bset, blk):
        s_, d_ = idxsets[bset]
        pltpu.async_copy(src_hbm.at[wid, pl.ds(blk * KBLK, KBLK)], s_, ib)
        pltpu.async_copy(dst_hbm.at[wid, pl.ds(blk * KBLK, KBLK)], d_, ib)

    def _stage_wait(bset, blk):
        s_, d_ = idxsets[bset]
        pltpu.make_async_copy(
            src_hbm.at[wid, pl.ds(blk * KBLK, KBLK)], s_, ib).wait()
        pltpu.make_async_copy(
            dst_hbm.at[wid, pl.ds(blk * KBLK, KBLK)], d_, ib).wait()

    def _g(s_, i, rb, gs):
        pltpu.async_copy(y_hbm.at[s_.at[i]], rb, gs)

    def _wg(s_, i, rb, gs):
        pltpu.make_async_copy(y_hbm.at[s_.at[i]], rb, gs).wait()

    def _s(d_, i, rb):
        pltpu.sync_copy(rb, acc_sh.at[d_.at[i]], add=True)

    _stage(0, 0)
    _stage_wait(0, 0)
    _stage(1, 1)

    sa, da = idxsets[0]
    for t in range(NBUF):
        _g(sa, t, rows[t], gsems[t])

    for t in range(40):
        pltpu.make_async_copy(zbuf_v, acc_sh.at[pl.ds(sid * 640, 16)],
                              zs).wait()
    plsc.subcore_barrier()

    for b in range(NBLK):
        s_, d_ = idxsets[b % 2]

        def _inner(i, _, s_=s_, d_=d_):
            j0 = NBUF * i
            for t in range(NBUF):
                _wg(s_, j0 + t, rows[t], gsems[t])
                _s(d_, j0 + t, rows[t])
                _g(s_, j0 + t + NBUF, rows[t], gsems[t])
            return 0
        lax.fori_loop(0, KBLK // NBUF - 1, _inner, 0)

        j0 = KBLK - NBUF
        if b < NBLK - 1:
            nset = (b + 1) % 2
            _stage_wait(nset, b + 1)
            sn, _dn = idxsets[nset]
            for t in range(NBUF):
                _wg(s_, j0 + t, rows[t], gsems[t])
                _s(d_, j0 + t, rows[t])
                _g(sn, t, rows[t], gsems[t])
            if b < NBLK - 2:
                _stage(b % 2, b + 2)
        else:
            for t in range(NBUF):
                _wg(s_, j0 + t, rows[t], gsems[t])
                _s(d_, j0 + t, rows[t])

    plsc.subcore_barrier()
    pltpu.sync_copy(acc_sh.at[pl.ds(sid * 640, 640)],
                    z_hbm.at[cid, pl.ds(sid * 640, 640)])


_sc_agg = functools.partial(
    pl.kernel,
    out_type=jax.ShapeDtypeStruct((2, NPAD, D), jnp.float32),
    mesh=_sc_mesh,
    scratch_types=[
        pltpu.VMEM((KBLK, CHUNK), jnp.int32),
        pltpu.VMEM((KBLK, CHUNK), jnp.int32),
        pltpu.VMEM((KBLK, CHUNK), jnp.int32),
        pltpu.VMEM((KBLK, CHUNK), jnp.int32),
        pltpu.VMEM((CHUNK, D), jnp.float32),
        pltpu.VMEM((CHUNK, D), jnp.float32),
        pltpu.VMEM((CHUNK, D), jnp.float32),
        pltpu.VMEM((CHUNK, D), jnp.float32),
        pltpu.VMEM((16, D), jnp.float32),
        pltpu.VMEM_SHARED((NPAD, D), jnp.float32),
        pltpu.SemaphoreType.DMA,
        pltpu.SemaphoreType.DMA,
        pltpu.SemaphoreType.DMA,
        pltpu.SemaphoreType.DMA,
        pltpu.SemaphoreType.DMA,
        pltpu.SemaphoreType.DMA,
    ],
)(_sc_agg_body)


def _scale_body(x_ref, d_ref, o_ref):
    i = pl.program_id(0)
    row = i * 1280 + lax.broadcasted_iota(jnp.int32, (1280, D), 0)
    o_ref[...] = jnp.where(row < N_NODES, x_ref[...] * d_ref[...], 0.0)


def _tc_scale(x, dinv_col):
    return pl.pallas_call(
        _scale_body,
        grid=(8,),
        in_specs=[
            pl.BlockSpec((1280, D), lambda i: (i, 0)),
            pl.BlockSpec((1280, 1), lambda i: (i, 0)),
        ],
        out_specs=pl.BlockSpec((1280, D), lambda i: (i, 0)),
        out_shape=jax.ShapeDtypeStruct((NPAD, D), jnp.float32),
    )(x, dinv_col)


def _head_body(z_ref, y_ref, d_ref, w_ref, b_ref, wih_ref, whh_ref, bs_ref,
               f1w_ref, f1b_ref, f2w_ref, f2b_ref, o_ref, gx_ref):
    i = pl.program_id(0)

    @pl.when(i < 8)
    def _gcn():
        zsum = z_ref[0] + z_ref[1] + y_ref[...]
        agg = zsum * d_ref[...]
        g = jnp.dot(agg, w_ref[...], preferred_element_type=jnp.float32)
        g = g + b_ref[...]
        h = jax.nn.relu(g[:, :HID]) + jax.nn.relu(g[:, HID:])
        gx_ref[pl.ds(i * 1280, 1280), :] = jnp.dot(
            h, wih_ref[...], preferred_element_type=jnp.float32)

    @pl.when(i == 8)
    def _lstm():
        whh = whh_ref[...]
        bs = bs_ref[...]

        def _sigm(v):
            return 1.0 / (1.0 + jnp.exp(-v))

        def _step(t, carry):
            hp, cp = carry
            gates = gx_ref[pl.ds(t * BATCH, BATCH), :]
            gates = gates + jnp.dot(hp, whh,
                                    preferred_element_type=jnp.float32)
            gates = gates + bs
            ig = _sigm(gates[:, 0 * HID:1 * HID])
            fg = _sigm(gates[:, 1 * HID:2 * HID])
            gg = jnp.tanh(gates[:, 2 * HID:3 * HID])
            og = _sigm(gates[:, 3 * HID:4 * HID])
            c = fg * cp + ig * gg
            hn = og * jnp.tanh(c)
            return (hn, c)

        h0 = jnp.zeros((BATCH, HID), jnp.float32)
        c0 = jnp.zeros((BATCH, HID), jnp.float32)
        hT, _ = lax.fori_loop(0, TSTEPS, _step, (h0, c0))
        u = jax.nn.relu(
            jnp.dot(hT, f1w_ref[...], preferred_element_type=jnp.float32)
            + f1b_ref[...])
        o_ref[...] = jnp.dot(u, f2w_ref[...],
                             preferred_element_type=jnp.float32) + f2b_ref[...]


def _tc_head(z, y, dinv_col, W12t, b12, Wiht, Whht, bsum,
             f1w, f1b, f2w, f2b):
    return pl.pallas_call(
        _head_body,
        grid=(9,),
        in_specs=[
            pl.BlockSpec((2, 1280, D), lambda i: (0, i % 8, 0)),
            pl.BlockSpec((1280, D), lambda i: (i % 8, 0)),
            pl.BlockSpec((1280, 1), lambda i: (i % 8, 0)),
            pl.BlockSpec((D, 2 * HID), lambda i: (0, 0)),
            pl.BlockSpec((1, 2 * HID), lambda i: (0, 0)),
            pl.BlockSpec((D, G4), lambda i: (0, 0)),
            pl.BlockSpec((D, G4), lambda i: (0, 0)),
            pl.BlockSpec((1, G4), lambda i: (0, 0)),
            pl.BlockSpec((D, HID2), lambda i: (0, 0)),
            pl.BlockSpec((1, HID2), lambda i: (0, 0)),
            pl.BlockSpec((HID2, D), lambda i: (0, 0)),
            pl.BlockSpec((1, D), lambda i: (0, 0)),
        ],
        out_specs=pl.BlockSpec((BATCH, HID), lambda i: (0, 0)),
        out_shape=jax.ShapeDtypeStruct((BATCH, HID), jnp.float32),
        scratch_shapes=[pltpu.VMEM((NPAD, G4), jnp.float32)],
    )(z, y, dinv_col, W12t, b12, Wiht, Whht, bsum, f1w, f1b, f2w, f2b)


def kernel(x, edge_index, W1, b1, W2, b2, Wih, Whh, bih, bhh,
           fc1W, fc1b, fc2W, fc2b):
    n_fill = E_PAD - E
    pad_idx = (N_NODES
               + jnp.arange(n_fill, dtype=jnp.int32) % (NPAD - N_NODES))
    src_flat = jnp.concatenate([edge_index[0], pad_idx])
    dst_flat = jnp.concatenate([edge_index[1], pad_idx])
    src_p = src_flat.reshape(NW, KCH, CHUNK)
    dst_p = dst_flat.reshape(NW, KCH, CHUNK)

    degp = _sc_deg(dst_flat.reshape(NW, DKCH, DCHUNK))
    deg = degp[0] + degp[1] + 1.0
    dinv_col = lax.rsqrt(deg)[:, None]

    y = _tc_scale(x, dinv_col)

    z = _sc_agg(y, src_p, dst_p)

    W12t = jnp.concatenate([W1.T, W2.T], axis=1)
    b12 = jnp.concatenate([b1, b2])[None]

    return _tc_head(z, y, dinv_col, W12t, b12, Wih.T, Whh.T,
                    (bih + bhh)[None], fc1W.T, fc1b[None],
                    fc2W.T, fc2b[None])


# speedup vs baseline: 1.0436x; 1.0058x over previous
import functools

import jax
import jax.numpy as jnp
from jax import lax
from jax.experimental import pallas as pl
from jax.experimental.pallas import tpu as pltpu
from jax.experimental.pallas import tpu_sc as plsc

N_NODES = 10000
D = 128
NPAD = 10240
E = 320000
NW = 32
CHUNK = 64
KCH = 160
KBLK = 16
NBLK = KCH // KBLK
NBUF = 4
DCHUNK = 128
DKCH = 80
E_PAD = NW * KCH * CHUNK
TSTEPS = 10
BATCH = 1000
HID = 128
HID2 = 64
G4 = 4 * HID

_sc_mesh = plsc.VectorSubcoreMesh(core_axis_name="c", subcore_axis_name="s")


def _sc_deg_body(dst_hbm, deg_hbm, idx_v, ones_v, zero_v, hist_sh, sd):
    cid = lax.axis_index("c")
    sid = lax.axis_index("s")
    wid = cid * 16 + sid

    ones16 = jnp.ones((16,), jnp.float32)
    zeros16 = jnp.zeros((16,), jnp.float32)
    for i in range(8):
        ones_v[pl.ds(i * 16, 16)] = ones16

    def _zfill(i, _):
        zero_v[pl.ds(i * 16, 16)] = zeros16
        return 0
    lax.fori_loop(0, 40, _zfill, 0)

    pltpu.sync_copy(zero_v, hist_sh.at[pl.ds(sid * 640, 640)])
    pltpu.sync_copy(dst_hbm.at[wid], idx_v)
    plsc.subcore_barrier()

    def _scat(j, _):
        pltpu.async_copy(ones_v, hist_sh.at[idx_v.at[j]], sd, add=True)
        return 0
    lax.fori_loop(0, DKCH, _scat, 0)

    def _drain(j, _):
        pltpu.make_async_copy(ones_v, hist_sh.at[idx_v.at[0]], sd).wait()
        return 0
    lax.fori_loop(0, DKCH, _drain, 0)

    plsc.subcore_barrier()
    pltpu.sync_copy(hist_sh.at[pl.ds(sid * 640, 640)],
                    deg_hbm.at[cid, pl.ds(sid * 640, 640)])


_sc_deg = functools.partial(
    pl.kernel,
    out_type=jax.ShapeDtypeStruct((2, NPAD), jnp.float32),
    mesh=_sc_mesh,
    scratch_types=[
        pltpu.VMEM((DKCH, DCHUNK), jnp.int32),
        pltpu.VMEM((DCHUNK,), jnp.float32),
        pltpu.VMEM((640,), jnp.float32),
        pltpu.VMEM_SHARED((NPAD,), jnp.float32),
        pltpu.SemaphoreType.DMA,
    ],
)(_sc_deg_body)


def _sc_agg_body(y_hbm, src_hbm, dst_hbm, z_hbm,
                 sA, dA, sB, dB, r0, r1, r2, r3, zbuf_v, acc_sh,
                 g0, g1, g2, g3, ib, zs):
    cid = lax.axis_index("c")
    sid = lax.axis_index("s")
    wid = cid * 16 + sid
    idxsets = ((sA, dA), (sB, dB))
    rows = (r0, r1, r2, r3)
    gsems = (g0, g1, g2, g3)

    zeros16 = jnp.zeros((16,), jnp.float32)

    def _zfill(i, _):
        for k in range(8):
            zbuf_v[i, pl.ds(k * 16, 16)] = zeros16
        return 0
    lax.fori_loop(0, 16, _zfill, 0)

    for t in range(40):
        pltpu.async_copy(zbuf_v, acc_sh.at[pl.ds(sid * 640 + t * 16, 16)], zs)

    def _stage(bset, blk):
        s_, d_ = idxsets[bset]
        pltpu.async_copy(src_hbm.at[wid, pl.ds(blk * KBLK, KBLK)], s_, ib)
        pltpu.async_copy(dst_hbm.at[wid, pl.ds(blk * KBLK, KBLK)], d_, ib)

    def _stage_wait(bset, blk):
        s_, d_ = idxsets[bset]
        pltpu.make_async_copy(
            src_hbm.at[wid, pl.ds(blk * KBLK, KBLK)], s_, ib).wait()
        pltpu.make_async_copy(
            dst_hbm.at[wid, pl.ds(blk * KBLK, KBLK)], d_, ib).wait()

    def _g(s_, i, rb, gs):
        pltpu.async_copy(y_hbm.at[s_.at[i]], rb, gs)

    def _wg(s_, i, rb, gs):
        pltpu.make_async_copy(y_hbm.at[s_.at[i]], rb, gs).wait()

    def _s(d_, i, rb):
        pltpu.sync_copy(rb, acc_sh.at[d_.at[i]], add=True)

    _stage(0, 0)
    _stage_wait(0, 0)
    _stage(1, 1)

    sa, da = idxsets[0]
    for t in range(NBUF):
        _g(sa, t, rows[t], gsems[t])

    for t in range(40):
        pltpu.make_async_copy(zbuf_v, acc_sh.at[pl.ds(sid * 640, 16)],
                              zs).wait()
    plsc.subcore_barrier()

    for b in range(NBLK):
        s_, d_ = idxsets[b % 2]

        def _inner(i, _, s_=s_, d_=d_):
            j0 = NBUF * i
            for t in range(NBUF):
                _wg(s_, j0 + t, rows[t], gsems[t])
                _s(d_, j0 + t, rows[t])
                _g(s_, j0 + t + NBUF, rows[t], gsems[t])
            return 0
        lax.fori_loop(0, KBLK // NBUF - 1, _inner, 0)

        j0 = KBLK - NBUF
        if b < NBLK - 1:
            nset = (b + 1) % 2
            _stage_wait(nset, b + 1)
            sn, _dn = idxsets[nset]
            for t in range(NBUF):
                _wg(s_, j0 + t, rows[t], gsems[t])
                _s(d_, j0 + t, rows[t])
                _g(sn, t, rows[t], gsems[t])
            if b < NBLK - 2:
                _stage(b % 2, b + 2)
        else:
            for t in range(NBUF):
                _wg(s_, j0 + t, rows[t], gsems[t])
                _s(d_, j0 + t, rows[t])

    plsc.subcore_barrier()
    pltpu.sync_copy(acc_sh.at[pl.ds(sid * 640, 640)],
                    z_hbm.at[cid, pl.ds(sid * 640, 640)])


_sc_agg = functools.partial(
    pl.kernel,
    out_type=jax.ShapeDtypeStruct((2, NPAD, D), jnp.float32),
    mesh=_sc_mesh,
    scratch_types=[
        pltpu.VMEM((KBLK, CHUNK), jnp.int32),
        pltpu.VMEM((KBLK, CHUNK), jnp.int32),
        pltpu.VMEM((KBLK, CHUNK), jnp.int32),
        pltpu.VMEM((KBLK, CHUNK), jnp.int32),
        pltpu.VMEM((CHUNK, D), jnp.float32),
        pltpu.VMEM((CHUNK, D), jnp.float32),
        pltpu.VMEM((CHUNK, D), jnp.float32),
        pltpu.VMEM((CHUNK, D), jnp.float32),
        pltpu.VMEM((16, D), jnp.float32),
        pltpu.VMEM_SHARED((NPAD, D), jnp.float32),
        pltpu.SemaphoreType.DMA,
        pltpu.SemaphoreType.DMA,
        pltpu.SemaphoreType.DMA,
        pltpu.SemaphoreType.DMA,
        pltpu.SemaphoreType.DMA,
        pltpu.SemaphoreType.DMA,
    ],
)(_sc_agg_body)


def _scale_body(x_ref, d_ref, o_ref):
    i = pl.program_id(0)
    row = i * 1280 + lax.broadcasted_iota(jnp.int32, (1280, D), 0)
    o_ref[...] = jnp.where(row < N_NODES, x_ref[...] * d_ref[...], 0.0)


def _tc_scale(x, dinv_col):
    return pl.pallas_call(
        _scale_body,
        grid=(8,),
        in_specs=[
            pl.BlockSpec((1280, D), lambda i: (i, 0)),
            pl.BlockSpec((1280, 1), lambda i: (i, 0)),
        ],
        out_specs=pl.BlockSpec((1280, D), lambda i: (i, 0)),
        out_shape=jax.ShapeDtypeStruct((NPAD, D), jnp.float32),
    )(x, dinv_col)


def _head_body(z_ref, y_ref, d_ref, w_ref, b_ref, wih_ref, whh_ref, bs_ref,
               f1w_ref, f1b_ref, f2w_ref, f2b_ref, o_ref, gx_ref):
    i = pl.program_id(0)

    @pl.when(i < 8)
    def _gcn():
        zsum = z_ref[0] + z_ref[1] + y_ref[...]
        agg = zsum * d_ref[...]
        g = jnp.dot(agg, w_ref[...], preferred_element_type=jnp.float32)
        g = g + b_ref[...]
        h = jax.nn.relu(g[:, :HID]) + jax.nn.relu(g[:, HID:])
        gx_ref[pl.ds(i * 1280, 1280), :] = jnp.dot(
            h, wih_ref[...], preferred_element_type=jnp.float32)

    @pl.when(i == 8)
    def _lstm():
        whh = whh_ref[...]
        bs = bs_ref[...]

        def _sigm(v):
            return 1.0 / (1.0 + jnp.exp(-v))

        def _step(t, carry):
            hp, cp = carry
            gates = gx_ref[pl.ds(t * BATCH, BATCH), :]
            gates = gates + jnp.dot(hp, whh,
                                    preferred_element_type=jnp.float32)
            gates = gates + bs
            ig = _sigm(gates[:, 0 * HID:1 * HID])
            fg = _sigm(gates[:, 1 * HID:2 * HID])
            gg = jnp.tanh(gates[:, 2 * HID:3 * HID])
            og = _sigm(gates[:, 3 * HID:4 * HID])
            c = fg * cp + ig * gg
            hn = og * jnp.tanh(c)
            return (hn, c)

        h0 = jnp.zeros((BATCH, HID), jnp.float32)
        c0 = jnp.zeros((BATCH, HID), jnp.float32)
        hT, _ = lax.fori_loop(0, TSTEPS, _step, (h0, c0))
        u = jax.nn.relu(
            jnp.dot(hT, f1w_ref[...], preferred_element_type=jnp.float32)
            + f1b_ref[...])
        o_ref[...] = jnp.dot(u, f2w_ref[...],
                             preferred_element_type=jnp.float32) + f2b_ref[...]


def _tc_head(z, y, dinv_col, W12t, b12, Wiht, Whht, bsum,
             f1w, f1b, f2w, f2b):
    return pl.pallas_call(
        _head_body,
        grid=(9,),
        in_specs=[
            pl.BlockSpec((2, 1280, D), lambda i: (0, jnp.minimum(i, 7), 0)),
            pl.BlockSpec((1280, D), lambda i: (jnp.minimum(i, 7), 0)),
            pl.BlockSpec((1280, 1), lambda i: (jnp.minimum(i, 7), 0)),
            pl.BlockSpec((D, 2 * HID), lambda i: (0, 0)),
            pl.BlockSpec((1, 2 * HID), lambda i: (0, 0)),
            pl.BlockSpec((D, G4), lambda i: (0, 0)),
            pl.BlockSpec((D, G4), lambda i: (0, 0)),
            pl.BlockSpec((1, G4), lambda i: (0, 0)),
            pl.BlockSpec((D, HID2), lambda i: (0, 0)),
            pl.BlockSpec((1, HID2), lambda i: (0, 0)),
            pl.BlockSpec((HID2, D), lambda i: (0, 0)),
            pl.BlockSpec((1, D), lambda i: (0, 0)),
        ],
        out_specs=pl.BlockSpec((BATCH, HID), lambda i: (0, 0)),
        out_shape=jax.ShapeDtypeStruct((BATCH, HID), jnp.float32),
        scratch_shapes=[pltpu.VMEM((NPAD, G4), jnp.float32)],
    )(z, y, dinv_col, W12t, b12, Wiht, Whht, bsum, f1w, f1b, f2w, f2b)


def kernel(x, edge_index, W1, b1, W2, b2, Wih, Whh, bih, bhh,
           fc1W, fc1b, fc2W, fc2b):
    n_fill = E_PAD - E
    pad_idx = (N_NODES
               + jnp.arange(n_fill, dtype=jnp.int32) % (NPAD - N_NODES))
    src_flat = jnp.concatenate([edge_index[0], pad_idx])
    dst_flat = jnp.concatenate([edge_index[1], pad_idx])
    src_p = src_flat.reshape(NW, KCH, CHUNK)
    dst_p = dst_flat.reshape(NW, KCH, CHUNK)

    degp = _sc_deg(dst_flat.reshape(NW, DKCH, DCHUNK))
    deg = degp[0] + degp[1] + 1.0
    dinv_col = lax.rsqrt(deg)[:, None]

    y = _tc_scale(x, dinv_col)

    z = _sc_agg(y, src_p, dst_p)

    W12t = jnp.concatenate([W1.T, W2.T], axis=1)
    b12 = jnp.concatenate([b1, b2])[None]

    return _tc_head(z, y, dinv_col, W12t, b12, Wih.T, Whh.T,
                    (bih + bhh)[None], fc1W.T, fc1b[None],
                    fc2W.T, fc2b[None])
